# baseline (speedup 1.0000x reference)
import jax
import jax.numpy as jnp
from jax import lax
from jax.experimental import pallas as pl
from jax.experimental.pallas import tpu as pltpu
from jax.experimental.pallas import tpu_sc as plsc

N_NODES = 10000
N_EDGES = 160000
D_IN = 55
D_EDGE = 6
HID = 256
HEADS = 4

N_PAD = 10240
E_PAD = 163840
NW = 32
EDGES_PER_W = E_PAD // NW
CHUNK = 4096
ROWBLK = 128
SENTINEL = 1 << 30
REC = 16

C1 = HEADS * HID
ROW1 = C1 + 16
C2 = HID
ROW2 = C2 + 16

_mesh = plsc.VectorSubcoreMesh(core_axis_name="c", subcore_axis_name="s")
_sc_params = pltpu.CompilerParams(needs_layout_passes=False)


def _tc1_body(x_ref, w_ref, as_ref, ad_ref, h_ref, nod_ref):
    xb = x_ref[...]
    h = jnp.dot(xb, w_ref[...], preferred_element_type=jnp.float32)
    h_ref[...] = h
    hh = h.reshape(ROWBLK, HEADS, HID)
    a_s = (hh * as_ref[...][None]).sum(-1)
    a_d = (hh * ad_ref[...][None]).sum(-1)
    nod_ref[...] = jnp.concatenate([a_s, a_d], axis=1)


def _tc1(x_pad, W1p, aS1, aD1):
    return pl.pallas_call(
        _tc1_body,
        grid=(N_PAD // ROWBLK,),
        in_specs=[
            pl.BlockSpec((ROWBLK, 64), lambda i: (i, 0)),
            pl.BlockSpec((64, C1), lambda i: (0, 0)),
            pl.BlockSpec((HEADS, HID), lambda i: (0, 0)),
            pl.BlockSpec((HEADS, HID), lambda i: (0, 0)),
        ],
        out_specs=[
            pl.BlockSpec((ROWBLK, C1), lambda i: (i, 0)),
            pl.BlockSpec((ROWBLK, 8), lambda i: (i, 0)),
        ],
        out_shape=[
            jax.ShapeDtypeStruct((N_PAD, C1), jnp.float32),
            jax.ShapeDtypeStruct((N_PAD, 8), jnp.float32),
        ],
    )(x_pad, W1p, aS1, aD1)


def _tce_body(w_ref, ea_ref, ae_ref):
    ae_ref[...] = jnp.dot(w_ref[...], ea_ref[...],
                          preferred_element_type=jnp.float32)


def _tce(WeApack, eaT_pad):
    return pl.pallas_call(
        _tce_body,
        grid=(E_PAD // 2048,),
        in_specs=[
            pl.BlockSpec((8, 8), lambda i: (0, 0)),
            pl.BlockSpec((8, 2048), lambda i: (0, i)),
        ],
        out_specs=pl.BlockSpec((8, 2048), lambda i: (0, i)),
        out_shape=jax.ShapeDtypeStruct((8, E_PAD), jnp.float32),
    )(WeApack, eaT_pad)


def _lrelu(x):
    return jnp.where(x >= 0, x, 0.2 * x)


def _post1_body(u_ref, h_ref, as_ref, ad_ref, b_ref, g_ref, be_ref,
                hn_ref, aux_ref):
    u = u_ref[...]
    h = h_ref[...]
    hh = h.reshape(ROWBLK, HEADS, HID)
    a_s = (hh * as_ref[...][None]).sum(-1)
    a_d = (hh * ad_ref[...][None]).sum(-1)
    den = u[:, C1:C1 + 4]
    sA1 = u[:, C1 + 4:C1 + 8]
    sA2 = u[:, C1 + 8:C1 + 9]
    cnt = u[:, C1 + 9:C1 + 10]
    cntc = jnp.maximum(cnt, 1.0)
    aes1 = sA1 / cntc
    exs = jnp.exp(_lrelu(a_s + a_d + aes1))
    numer = u[:, :C1].reshape(ROWBLK, HEADS, HID) + hh * exs[:, :, None]
    o = (numer / (den + exs)[:, :, None]).reshape(ROWBLK, C1) + b_ref[...]
    m = o.mean(-1, keepdims=True)
    v = ((o - m) ** 2).mean(-1, keepdims=True)
    on = (o - m) / jnp.sqrt(v + 1e-5) * g_ref[...] + be_ref[...]
    hn_ref[...] = jnp.maximum(on, 0.0)
    aux_ref[...] = jnp.concatenate(
        [sA2 / cntc, jnp.zeros((ROWBLK, 7), jnp.float32)], axis=1)


def _post1(out_un1, h1, aS1, aD1, b1, g1, be1):
    return pl.pallas_call(
        _post1_body,
        grid=(N_PAD // ROWBLK,),
        in_specs=[
            pl.BlockSpec((ROWBLK, ROW1), lambda i: (i, 0)),
            pl.BlockSpec((ROWBLK, C1), lambda i: (i, 0)),
            pl.BlockSpec((HEADS, HID), lambda i: (0, 0)),
            pl.BlockSpec((HEADS, HID), lambda i: (0, 0)),
            pl.BlockSpec((1, C1), lambda i: (0, 0)),
            pl.BlockSpec((1, C1), lambda i: (0, 0)),
            pl.BlockSpec((1, C1), lambda i: (0, 0)),
        ],
        out_specs=[
            pl.BlockSpec((ROWBLK, C1), lambda i: (i, 0)),
            pl.BlockSpec((ROWBLK, 8), lambda i: (i, 0)),
        ],
        out_shape=[
            jax.ShapeDtypeStruct((N_PAD, C1), jnp.float32),
            jax.ShapeDtypeStruct((N_PAD, 8), jnp.float32),
        ],
    )(out_un1, h1, aS1, aD1, b1, g1, be1)


def _tc2_body(h_ref, w_ref, aux_ref, h2_ref, nod_ref):
    hx = jnp.dot(h_ref[...], w_ref[...], preferred_element_type=jnp.float32)
    h2_ref[...] = hx[:, :C2]
    a_s = hx[:, C2:C2 + 1]
    a_d = hx[:, C2 + 1:C2 + 2]
    exs = jnp.exp(_lrelu(a_s + a_d + aux_ref[:, 0:1]))
    nod_ref[...] = jnp.concatenate(
        [a_s, a_d, exs, jnp.zeros((ROWBLK, 5), jnp.float32)], axis=1)


def _tc2(h1n, W2aug, aux1):
    return pl.pallas_call(
        _tc2_body,
        grid=(N_PAD // ROWBLK,),
        in_specs=[
            pl.BlockSpec((ROWBLK, C1), lambda i: (i, 0)),
            pl.BlockSpec((C1, 384), lambda i: (0, 0)),
            pl.BlockSpec((ROWBLK, 8), lambda i: (i, 0)),
        ],
        out_specs=[
            pl.BlockSpec((ROWBLK, C2), lambda i: (i, 0)),
            pl.BlockSpec((ROWBLK, 8), lambda i: (i, 0)),
        ],
        out_shape=[
            jax.ShapeDtypeStruct((N_PAD, C2), jnp.float32),
            jax.ShapeDtypeStruct((N_PAD, 8), jnp.float32),
        ],
    )(h1n, W2aug, aux1)


def _post2_body(u_ref, h_ref, nod_ref, b_ref, g_ref, be_ref, o_ref):
    u = u_ref[...]
    h = h_ref[...]
    den = u[:, C2:C2 + 1]
    exs = nod_ref[:, 2:3]
    o = (u[:, :C2] + h * exs) / (den + exs) + b_ref[...]
    m = o.mean(-1, keepdims=True)
    v = ((o - m) ** 2).mean(-1, keepdims=True)
    o_ref[...] = (o - m) / jnp.sqrt(v + 1e-5) * g_ref[...] + be_ref[...]


def _post2(out_un2, h2t, nod2, b2, g2, be2):
    return pl.pallas_call(
        _post2_body,
        grid=(N_PAD // ROWBLK,),
        in_specs=[
            pl.BlockSpec((ROWBLK, ROW2), lambda i: (i, 0)),
            pl.BlockSpec((ROWBLK, C2), lambda i: (i, 0)),
            pl.BlockSpec((ROWBLK, 8), lambda i: (i, 0)),
            pl.BlockSpec((1, C2), lambda i: (0, 0)),
            pl.BlockSpec((1, C2), lambda i: (0, 0)),
            pl.BlockSpec((1, C2), lambda i: (0, 0)),
        ],
        out_specs=pl.BlockSpec((ROWBLK, C2), lambda i: (i, 0)),
        out_shape=jax.ShapeDtypeStruct((N_PAD, C2), jnp.float32),
    )(out_un2, h2t, nod2, b2, g2, be2)


BCHUNK = 128


def _phaseB_make(heads, src_cols, dst_cols, ae_rows, with_ae):
    n_groups = BCHUNK // 16
    n_chunks = EDGES_PER_W // BCHUNK
    nload = heads + (1 if with_ae else 0)

    def body(nod_hbm, src_hbm, dst_hbm, aeT_hbm, rec_hbm,
             nod_v, src_v, dst_v, ae_v, rec_v):
        wid = lax.axis_index("s") * 2 + lax.axis_index("c")
        pltpu.sync_copy(nod_hbm, nod_v)

        for g in range(8):
            @pl.loop(0, BCHUNK)
            def _z(i, _g=g):
                rec_v[pl.ds(i * 128 + _g * 16, 16)] = (
                    jnp.zeros((16,), jnp.float32))

        @pl.loop(0, n_chunks)
        def _chunk(c):
            base = wid * EDGES_PER_W + c * BCHUNK
            pltpu.sync_copy(src_hbm.at[pl.ds(base, BCHUNK)], src_v)
            pltpu.sync_copy(dst_hbm.at[pl.ds(base, BCHUNK)], dst_v)
            for h in range(nload):
                row = ae_rows[h] if h < heads else HEADS
                pltpu.sync_copy(
                    aeT_hbm.at[pl.ds(row * E_PAD + base, BCHUNK)],
                    ae_v.at[pl.ds(h * BCHUNK, BCHUNK)])

            @pl.loop(0, n_groups)
            def _grp(g):
                s16 = src_v[pl.ds(g * 16, 16)]
                d16 = dst_v[pl.ds(g * 16, 16)]
                s8 = s16 * 8
                d8 = d16 * 8
                lanes = lax.iota(jnp.int32, 16)
                real = (base + g * 16 + lanes) < N_EDGES
                zero = jnp.zeros((16,), jnp.float32)
                ridx = (lanes + g * 16) * 128
                for h in range(heads):
                    asr = plsc.load_gather(nod_v, [s8 + src_cols[h]])
                    ads = plsc.load_gather(nod_v, [d8 + dst_cols[h]])
                    ae16 = ae_v[pl.ds(h * BCHUNK + g * 16, 16)]
                    ex = jnp.exp(_lrelu(asr + ads + ae16))
                    plsc.store_scatter(rec_v, [ridx + h],
                                       jnp.where(real, ex, zero))
                    if with_ae:
                        plsc.store_scatter(rec_v, [ridx + heads + h],
                                           jnp.where(real, ae16, zero))
                if with_ae:
                    ae2 = ae_v[pl.ds(heads * BCHUNK + g * 16, 16)]
                    plsc.store_scatter(rec_v, [ridx + 2 * heads],
                                       jnp.where(real, ae2, zero))
                    plsc.store_scatter(
                        rec_v, [ridx + 2 * heads + 1],
                        jnp.where(real, jnp.full((16,), 1.0, jnp.float32),
                                  zero))
                plsc.store_scatter(rec_v, [ridx + 10],
                                   plsc.bitcast(s16, jnp.float32))

            pltpu.sync_copy(rec_v, rec_hbm.at[pl.ds(base * 128, BCHUNK * 128)])

    return pl.kernel(
        body,
        out_type=jax.ShapeDtypeStruct((E_PAD * 128,), jnp.float32),
        mesh=_mesh,
        compiler_params=_sc_params,
        scratch_types=[
            pltpu.VMEM((N_PAD * 8,), jnp.float32),
            pltpu.VMEM((BCHUNK,), jnp.int32),
            pltpu.VMEM((BCHUNK,), jnp.int32),
            pltpu.VMEM((max(nload, 1) * BCHUNK,), jnp.float32),
            pltpu.VMEM((BCHUNK * 128,), jnp.float32),
        ],
    )


_phaseB1 = _phaseB_make(HEADS, [0, 1, 2, 3], [4, 5, 6, 7], [0, 1, 2, 3], True)
_phaseB2 = _phaseB_make(1, [0], [1], [HEADS], False)


def _phaseC_make(cdim, heads, row_w, rows_tec, n_sweep):
    n_groups = CHUNK // 16
    n_chunks = E_PAD // CHUNK
    vregs_per_head = cdim // heads // 16
    nvr = row_w // 16
    LCAP = 6144 + 64
    THRESH = LCAP - CHUNK - 64
    lbits = (rows_tec - 1).bit_length()
    lmask = (1 << lbits) - 1

    def body(h_hbm, dst_hbm, rec_hbm, out_hbm,
             dst_v, eid_l, rec0, rec1, hold0, hold1,
             gath0, gath1, acc, rsem0, rsem1, gsem0, gsem1):
        cid = lax.axis_index("c")
        sid = lax.axis_index("s")
        wid = sid * 2 + cid
        iota16 = lax.iota(jnp.int32, 16)
        c10 = jnp.full((16,), 10, jnp.int32)

        def _issue_rec(b, rec, rsem):
            evec = lax.shift_right_logical(eid_l[pl.ds(b * 16, 16)], lbits)
            return pltpu.async_copy(rec_hbm.at[evec], rec, rsem)

        def _consume_rec(rec, rsem, hold, gath, gsem):
            pltpu.make_async_copy(rec_hbm.at[pl.ds(0, 16)], rec, rsem).wait()
            gvec = plsc.bitcast(plsc.load_gather(rec, [iota16, c10]),
                                jnp.int32)
            for j in range(16):
                hold[pl.ds(j * 16, 16)] = rec[j, pl.ds(0, 16)]
            return pltpu.async_copy(h_hbm.at[gvec], gath, gsem)

        def _process(b, hold, gath, gsem):
            pltpu.make_async_copy(h_hbm.at[pl.ds(0, 16)], gath, gsem).wait()

            @pl.loop(0, 16)
            def _scale(j):
                lrow = eid_l[pl.ds(b * 16 + j, 16)][0] & lmask
                for h in range(heads):
                    sc = plsc.load_gather(hold, [jnp.full((16,), j * 16 + h, jnp.int32)])
                    for i in range(vregs_per_head):
                        off = (h * vregs_per_head + i) * 16
                        plsc.addupdate(acc.at[lrow, pl.ds(off, 16)],
                                       gath[j, pl.ds(off, 16)] * sc)
                plsc.addupdate(acc.at[lrow, pl.ds(cdim, 16)],
                               hold[pl.ds(j * 16, 16)])

        def _drain_serial(nb):
            @pl.loop(0, nb)
            def _fire(fb):
                _issue_rec(fb, rec0, rsem0)
                _consume_rec(rec0, rsem0, hold0, gath0, gsem0).wait()

                @pl.loop(0, 16)
                def _scale(j):
                    lrow = eid_l[pl.ds(fb * 16 + j, 16)][0] & lmask
                    for h in range(heads):
                        sc = plsc.load_gather(hold0, [jnp.full((16,), j * 16 + h, jnp.int32)])
                        for i in range(vregs_per_head):
                            off = (h * vregs_per_head + i) * 16
                            plsc.addupdate(acc.at[lrow, pl.ds(off, 16)],
                                           gath0[j, pl.ds(off, 16)] * sc)
                    plsc.addupdate(acc.at[lrow, pl.ds(cdim, 16)],
                                   hold0[pl.ds(j * 16, 16)])

        def _drain_piped(nb):
            npair = nb // 2

            @pl.when(nb > 0)
            def _():
                _issue_rec(0, rec0, rsem0)
                _issue_rec(1, rec1, rsem1)

            @pl.loop(0, npair)
            def _pair(k):
                b0 = 2 * k
                _consume_rec(rec0, rsem0, hold0, gath0, gsem0)

                @pl.when(b0 + 2 < nb)
                def _():
                    _issue_rec(b0 + 2, rec0, rsem0)

                @pl.when(k > 0)
                def _():
                    _process(b0 - 1, hold1, gath1, gsem1)

                _consume_rec(rec1, rsem1, hold1, gath1, gsem1)

                @pl.when(b0 + 3 < nb)
                def _():
                    _issue_rec(b0 + 3, rec1, rsem1)

                _process(b0, hold0, gath0, gsem0)

            @pl.when(nb > 0)
            def _():
                _process(nb - 1, hold1, gath1, gsem1)

        @pl.loop(0, n_sweep)
        def _sweep(s):
            lo = (s * 32 + wid) * rows_tec

            @pl.loop(0, rows_tec)
            def _zr(r):
                @pl.loop(0, nvr)
                def _zc(cc, _r=r):
                    acc[_r, pl.ds(cc * 16, 16)] = jnp.zeros((16,),
                                                            jnp.float32)

            def _chunk(c, cnt):
                base = c * CHUNK
                pltpu.sync_copy(dst_hbm.at[pl.ds(base, CHUNK)], dst_v)

                def _grp(g, cnt):
                    d16 = dst_v[pl.ds(g * 16, 16)]
                    ld = d16 - lo
                    m = (ld >= 0) & (ld < rows_tec)
                    pk = lax.shift_left(base + g * 16 + iota16, lbits) | ld
                    plsc.store_compressed(eid_l.at[pl.ds(cnt, 16)], pk,
                                          mask=m)
                    pc = plsc.all_reduce_population_count(m)
                    return cnt + pc[0]

                cnt = pl.loop(0, n_groups, init_carry=cnt, unroll=4)(_grp)

                nb_mid = jnp.where(cnt >= THRESH, cnt // 16, 0)
                _drain_serial(nb_mid)
                ev = eid_l[pl.ds(nb_mid * 16, 16)]
                eid_l[pl.ds(0, 16)] = ev
                return cnt - nb_mid * 16

            cnt = pl.loop(0, n_chunks, init_carry=jnp.int32(0))(_chunk)

            padv = jnp.full((16,), N_EDGES << lbits, jnp.int32)
            eid_l[pl.ds(cnt, 16)] = padv
            eid_l[pl.ds(cnt + 16, 16)] = padv
            _drain_piped(((cnt + 31) // 32) * 2)

            pltpu.sync_copy(acc, out_hbm.at[pl.ds(lo, rows_tec)])

    return pl.kernel(
        body,
        out_type=jax.ShapeDtypeStruct((N_PAD, row_w), jnp.float32),
        mesh=_mesh,
        compiler_params=_sc_params,
        scratch_types=[
            pltpu.VMEM((CHUNK,), jnp.int32),
            pltpu.VMEM((LCAP,), jnp.int32),
            pltpu.VMEM((16, 128), jnp.float32),
            pltpu.VMEM((16, 128), jnp.float32),
            pltpu.VMEM((256,), jnp.float32),
            pltpu.VMEM((256,), jnp.float32),
            pltpu.VMEM((16, cdim), jnp.float32),
            pltpu.VMEM((16, cdim), jnp.float32),
            pltpu.VMEM((rows_tec, row_w), jnp.float32),
            pltpu.SemaphoreType.DMA,
            pltpu.SemaphoreType.DMA,
            pltpu.SemaphoreType.DMA,
            pltpu.SemaphoreType.DMA,
        ],
    )


_phaseC1 = _phaseC_make(C1, HEADS, ROW1, 64, 5)
_phaseC2 = _phaseC_make(C2, 1, ROW2, 160, 2)


def kernel(x, edge_index, edge_attr, W1, aS1, aD1, We1, aE1, b1, g1, be1,
           W2, aS2, aD2, We2, aE2, b2, g2, be2):
    f32 = jnp.float32
    src = edge_index[0].astype(jnp.int32)
    dst = edge_index[1].astype(jnp.int32)

    x_pad = jnp.zeros((N_PAD, 64), f32).at[:N_NODES, :D_IN].set(x)
    W1p = jnp.zeros((64, C1), f32).at[:D_IN].set(W1)
    e_pad = E_PAD - N_EDGES
    srcB = jnp.concatenate([src, jnp.zeros((e_pad,), jnp.int32)])
    dstB = jnp.concatenate([dst, jnp.zeros((e_pad,), jnp.int32)])
    dstC = jnp.concatenate([dst, jnp.full((e_pad,), SENTINEL, jnp.int32)])
    eaT_pad = jnp.zeros((8, E_PAD), f32).at[:D_EDGE, :N_EDGES].set(
        edge_attr.T)
    WeA1 = (We1.reshape(D_EDGE, HEADS, HID) * aE1).sum(-1)
    WeA2 = (We2.reshape(D_EDGE, 1, HID) * aE2).sum(-1)
    WeApack = jnp.zeros((8, 8), f32)
    WeApack = WeApack.at[:HEADS, :D_EDGE].set(WeA1.T)
    WeApack = WeApack.at[HEADS, :D_EDGE].set(WeA2[:, 0])
    WaS2 = (W2.reshape(C1, 1, HID) * aS2).sum(-1)
    WaD2 = (W2.reshape(C1, 1, HID) * aD2).sum(-1)
    W2aug = jnp.zeros((C1, 384), f32)
    W2aug = W2aug.at[:, :C2].set(W2)
    W2aug = W2aug.at[:, C2:C2 + 1].set(WaS2)
    W2aug = W2aug.at[:, C2 + 1:C2 + 2].set(WaD2)

    aS1r = aS1[0]
    aD1r = aD1[0]
    b1r = b1.reshape(1, C1)
    g1r = g1.reshape(1, C1)
    be1r = be1.reshape(1, C1)
    b2r = b2.reshape(1, C2)
    g2r = g2.reshape(1, C2)
    be2r = be2.reshape(1, C2)

    h1, nod1 = _tc1(x_pad, W1p, aS1r, aD1r)
    aeT = _tce(WeApack, eaT_pad).reshape(-1)
    rec1 = _phaseB1(nod1.reshape(-1), srcB, dstB, aeT).reshape(E_PAD, 128)
    out_un1 = _phaseC1(h1, dstC, rec1)
    h1n, aux1 = _post1(out_un1, h1, aS1r, aD1r, b1r, g1r, be1r)

    h2t, nod2 = _tc2(h1n, W2aug, aux1)
    rec2 = _phaseB2(nod2.reshape(-1), srcB, dstB, aeT).reshape(E_PAD, 128)
    out_un2 = _phaseC2(h2t, dstC, rec2)
    out = _post2(out_un2, h2t, nod2, b2r, g2r, be2r)

    return out[:N_NODES]


# speedup vs baseline: 1.0093x; 1.0093x over previous
import jax
import jax.numpy as jnp
from jax import lax
from jax.experimental import pallas as pl
from jax.experimental.pallas import tpu as pltpu
from jax.experimental.pallas import tpu_sc as plsc

N_NODES = 10000
N_EDGES = 160000
D_IN = 55
D_EDGE = 6
HID = 256
HEADS = 4

N_PAD = 10240
E_PAD = 163840
NW = 32
EDGES_PER_W = E_PAD // NW
CHUNK = 4096
ROWBLK = 128
SENTINEL = 1 << 30
REC = 16

C1 = HEADS * HID
ROW1 = C1 + 16
C2 = HID
ROW2 = C2 + 16

_mesh = plsc.VectorSubcoreMesh(core_axis_name="c", subcore_axis_name="s")
_sc_params = pltpu.CompilerParams(needs_layout_passes=False)


def _tc1_body(x_ref, w_ref, as_ref, ad_ref, h_ref, nod_ref):
    xb = x_ref[...]
    h = jnp.dot(xb, w_ref[...], preferred_element_type=jnp.float32)
    h_ref[...] = h
    hh = h.reshape(ROWBLK, HEADS, HID)
    a_s = (hh * as_ref[...][None]).sum(-1)
    a_d = (hh * ad_ref[...][None]).sum(-1)
    nod_ref[...] = jnp.concatenate([a_s, a_d], axis=1)


def _tc1(x_pad, W1p, aS1, aD1):
    return pl.pallas_call(
        _tc1_body,
        grid=(N_PAD // ROWBLK,),
        in_specs=[
            pl.BlockSpec((ROWBLK, 64), lambda i: (i, 0)),
            pl.BlockSpec((64, C1), lambda i: (0, 0)),
            pl.BlockSpec((HEADS, HID), lambda i: (0, 0)),
            pl.BlockSpec((HEADS, HID), lambda i: (0, 0)),
        ],
        out_specs=[
            pl.BlockSpec((ROWBLK, C1), lambda i: (i, 0)),
            pl.BlockSpec((ROWBLK, 8), lambda i: (i, 0)),
        ],
        out_shape=[
            jax.ShapeDtypeStruct((N_PAD, C1), jnp.float32),
            jax.ShapeDtypeStruct((N_PAD, 8), jnp.float32),
        ],
    )(x_pad, W1p, aS1, aD1)


def _tce_body(w_ref, ea_ref, ae_ref):
    ae_ref[...] = jnp.dot(w_ref[...], ea_ref[...],
                          preferred_element_type=jnp.float32)


def _tce(WeApack, eaT_pad):
    return pl.pallas_call(
        _tce_body,
        grid=(E_PAD // 2048,),
        in_specs=[
            pl.BlockSpec((8, 8), lambda i: (0, 0)),
            pl.BlockSpec((8, 2048), lambda i: (0, i)),
        ],
        out_specs=pl.BlockSpec((8, 2048), lambda i: (0, i)),
        out_shape=jax.ShapeDtypeStruct((8, E_PAD), jnp.float32),
    )(WeApack, eaT_pad)


def _lrelu(x):
    return jnp.where(x >= 0, x, 0.2 * x)


def _post1_body(u_ref, h_ref, as_ref, ad_ref, b_ref, g_ref, be_ref,
                hn_ref, aux_ref):
    u = u_ref[...]
    h = h_ref[...]
    hh = h.reshape(ROWBLK, HEADS, HID)
    a_s = (hh * as_ref[...][None]).sum(-1)
    a_d = (hh * ad_ref[...][None]).sum(-1)
    den = u[:, C1:C1 + 4]
    sA1 = u[:, C1 + 4:C1 + 8]
    sA2 = u[:, C1 + 8:C1 + 9]
    cnt = u[:, C1 + 9:C1 + 10]
    cntc = jnp.maximum(cnt, 1.0)
    aes1 = sA1 / cntc
    exs = jnp.exp(_lrelu(a_s + a_d + aes1))
    numer = u[:, :C1].reshape(ROWBLK, HEADS, HID) + hh * exs[:, :, None]
    o = (numer / (den + exs)[:, :, None]).reshape(ROWBLK, C1) + b_ref[...]
    m = o.mean(-1, keepdims=True)
    v = ((o - m) ** 2).mean(-1, keepdims=True)
    on = (o - m) / jnp.sqrt(v + 1e-5) * g_ref[...] + be_ref[...]
    hn_ref[...] = jnp.maximum(on, 0.0)
    aux_ref[...] = jnp.concatenate(
        [sA2 / cntc, jnp.zeros((ROWBLK, 7), jnp.float32)], axis=1)


def _post1(out_un1, h1, aS1, aD1, b1, g1, be1):
    return pl.pallas_call(
        _post1_body,
        grid=(N_PAD // ROWBLK,),
        in_specs=[
            pl.BlockSpec((ROWBLK, ROW1), lambda i: (i, 0)),
            pl.BlockSpec((ROWBLK, C1), lambda i: (i, 0)),
            pl.BlockSpec((HEADS, HID), lambda i: (0, 0)),
            pl.BlockSpec((HEADS, HID), lambda i: (0, 0)),
            pl.BlockSpec((1, C1), lambda i: (0, 0)),
            pl.BlockSpec((1, C1), lambda i: (0, 0)),
            pl.BlockSpec((1, C1), lambda i: (0, 0)),
        ],
        out_specs=[
            pl.BlockSpec((ROWBLK, C1), lambda i: (i, 0)),
            pl.BlockSpec((ROWBLK, 8), lambda i: (i, 0)),
        ],
        out_shape=[
            jax.ShapeDtypeStruct((N_PAD, C1), jnp.float32),
            jax.ShapeDtypeStruct((N_PAD, 8), jnp.float32),
        ],
    )(out_un1, h1, aS1, aD1, b1, g1, be1)


def _tc2_body(h_ref, w_ref, aux_ref, h2_ref, nod_ref):
    hx = jnp.dot(h_ref[...], w_ref[...], preferred_element_type=jnp.float32)
    h2_ref[...] = hx[:, :C2]
    a_s = hx[:, C2:C2 + 1]
    a_d = hx[:, C2 + 1:C2 + 2]
    exs = jnp.exp(_lrelu(a_s + a_d + aux_ref[:, 0:1]))
    nod_ref[...] = jnp.concatenate(
        [a_s, a_d, exs, jnp.zeros((ROWBLK, 5), jnp.float32)], axis=1)


def _tc2(h1n, W2aug, aux1):
    return pl.pallas_call(
        _tc2_body,
        grid=(N_PAD // ROWBLK,),
        in_specs=[
            pl.BlockSpec((ROWBLK, C1), lambda i: (i, 0)),
            pl.BlockSpec((C1, 384), lambda i: (0, 0)),
            pl.BlockSpec((ROWBLK, 8), lambda i: (i, 0)),
        ],
        out_specs=[
            pl.BlockSpec((ROWBLK, C2), lambda i: (i, 0)),
            pl.BlockSpec((ROWBLK, 8), lambda i: (i, 0)),
        ],
        out_shape=[
            jax.ShapeDtypeStruct((N_PAD, C2), jnp.float32),
            jax.ShapeDtypeStruct((N_PAD, 8), jnp.float32),
        ],
    )(h1n, W2aug, aux1)


def _post2_body(u_ref, h_ref, nod_ref, b_ref, g_ref, be_ref, o_ref):
    u = u_ref[...]
    h = h_ref[...]
    den = u[:, C2:C2 + 1]
    exs = nod_ref[:, 2:3]
    o = (u[:, :C2] + h * exs) / (den + exs) + b_ref[...]
    m = o.mean(-1, keepdims=True)
    v = ((o - m) ** 2).mean(-1, keepdims=True)
    o_ref[...] = (o - m) / jnp.sqrt(v + 1e-5) * g_ref[...] + be_ref[...]


def _post2(out_un2, h2t, nod2, b2, g2, be2):
    return pl.pallas_call(
        _post2_body,
        grid=(N_PAD // ROWBLK,),
        in_specs=[
            pl.BlockSpec((ROWBLK, ROW2), lambda i: (i, 0)),
            pl.BlockSpec((ROWBLK, C2), lambda i: (i, 0)),
            pl.BlockSpec((ROWBLK, 8), lambda i: (i, 0)),
            pl.BlockSpec((1, C2), lambda i: (0, 0)),
            pl.BlockSpec((1, C2), lambda i: (0, 0)),
            pl.BlockSpec((1, C2), lambda i: (0, 0)),
        ],
        out_specs=pl.BlockSpec((ROWBLK, C2), lambda i: (i, 0)),
        out_shape=jax.ShapeDtypeStruct((N_PAD, C2), jnp.float32),
    )(out_un2, h2t, nod2, b2, g2, be2)


BCHUNK = 128


def _phaseB_make(heads, src_cols, dst_cols, ae_rows, with_ae):
    n_groups = BCHUNK // 16
    n_chunks = EDGES_PER_W // BCHUNK
    nload = heads + (1 if with_ae else 0)

    def body(nod_hbm, src_hbm, dst_hbm, aeT_hbm, rec_hbm,
             nod_v, src_v, dst_v, ae_v, rec_v):
        wid = lax.axis_index("s") * 2 + lax.axis_index("c")
        pltpu.sync_copy(nod_hbm, nod_v)

        for g in range(8):
            @pl.loop(0, BCHUNK)
            def _z(i, _g=g):
                rec_v[pl.ds(i * 128 + _g * 16, 16)] = (
                    jnp.zeros((16,), jnp.float32))

        @pl.loop(0, n_chunks)
        def _chunk(c):
            base = wid * EDGES_PER_W + c * BCHUNK
            pltpu.sync_copy(src_hbm.at[pl.ds(base, BCHUNK)], src_v)
            pltpu.sync_copy(dst_hbm.at[pl.ds(base, BCHUNK)], dst_v)
            for h in range(nload):
                row = ae_rows[h] if h < heads else HEADS
                pltpu.sync_copy(
                    aeT_hbm.at[pl.ds(row * E_PAD + base, BCHUNK)],
                    ae_v.at[pl.ds(h * BCHUNK, BCHUNK)])

            @pl.loop(0, n_groups)
            def _grp(g):
                s16 = src_v[pl.ds(g * 16, 16)]
                d16 = dst_v[pl.ds(g * 16, 16)]
                s8 = s16 * 8
                d8 = d16 * 8
                lanes = lax.iota(jnp.int32, 16)
                real = (base + g * 16 + lanes) < N_EDGES
                zero = jnp.zeros((16,), jnp.float32)
                ridx = (lanes + g * 16) * 128
                for h in range(heads):
                    asr = plsc.load_gather(nod_v, [s8 + src_cols[h]])
                    ads = plsc.load_gather(nod_v, [d8 + dst_cols[h]])
                    ae16 = ae_v[pl.ds(h * BCHUNK + g * 16, 16)]
                    ex = jnp.exp(_lrelu(asr + ads + ae16))
                    plsc.store_scatter(rec_v, [ridx + h],
                                       jnp.where(real, ex, zero))
                    if with_ae:
                        plsc.store_scatter(rec_v, [ridx + heads + h],
                                           jnp.where(real, ae16, zero))
                if with_ae:
                    ae2 = ae_v[pl.ds(heads * BCHUNK + g * 16, 16)]
                    plsc.store_scatter(rec_v, [ridx + 2 * heads],
                                       jnp.where(real, ae2, zero))
                    plsc.store_scatter(
                        rec_v, [ridx + 2 * heads + 1],
                        jnp.where(real, jnp.full((16,), 1.0, jnp.float32),
                                  zero))
                plsc.store_scatter(rec_v, [ridx + 10],
                                   plsc.bitcast(s16, jnp.float32))

            pltpu.sync_copy(rec_v, rec_hbm.at[pl.ds(base * 128, BCHUNK * 128)])

    return pl.kernel(
        body,
        out_type=jax.ShapeDtypeStruct((E_PAD * 128,), jnp.float32),
        mesh=_mesh,
        compiler_params=_sc_params,
        scratch_types=[
            pltpu.VMEM((N_PAD * 8,), jnp.float32),
            pltpu.VMEM((BCHUNK,), jnp.int32),
            pltpu.VMEM((BCHUNK,), jnp.int32),
            pltpu.VMEM((max(nload, 1) * BCHUNK,), jnp.float32),
            pltpu.VMEM((BCHUNK * 128,), jnp.float32),
        ],
    )


_phaseB1 = _phaseB_make(HEADS, [0, 1, 2, 3], [4, 5, 6, 7], [0, 1, 2, 3], True)
_phaseB2 = _phaseB_make(1, [0], [1], [HEADS], False)


def _phaseC_make(cdim, heads, row_w, rows_tec, n_sweep):
    n_groups = CHUNK // 16
    n_chunks = E_PAD // CHUNK
    vregs_per_head = cdim // heads // 16
    nvr = row_w // 16
    LCAP = 6144 + 64
    THRESH = LCAP - CHUNK - 64

    def body(h_hbm, dst_hbm, rec_hbm, out_hbm,
             dst_v, eid_l, ldst_l, rec0, rec1, hold0, hold1,
             gath0, gath1, acc, rsem0, rsem1, gsem0, gsem1):
        cid = lax.axis_index("c")
        sid = lax.axis_index("s")
        wid = sid * 2 + cid
        iota16 = lax.iota(jnp.int32, 16)
        c10 = jnp.full((16,), 10, jnp.int32)

        def _issue_rec(b, rec, rsem):
            evec = eid_l[pl.ds(b * 16, 16)]
            return pltpu.async_copy(rec_hbm.at[evec], rec, rsem)

        def _consume_rec(rec, rsem, hold, gath, gsem):
            pltpu.make_async_copy(rec_hbm.at[pl.ds(0, 16)], rec, rsem).wait()
            gvec = plsc.bitcast(plsc.load_gather(rec, [iota16, c10]),
                                jnp.int32)
            for j in range(16):
                hold[pl.ds(j * 16, 16)] = rec[j, pl.ds(0, 16)]
            return pltpu.async_copy(h_hbm.at[gvec], gath, gsem)

        def _process(b, hold, gath, gsem):
            pltpu.make_async_copy(h_hbm.at[pl.ds(0, 16)], gath, gsem).wait()

            @pl.loop(0, 16)
            def _scale(j):
                lrow = ldst_l[pl.ds(b * 16 + j, 16)][0]
                for h in range(heads):
                    sc = plsc.load_gather(hold, [jnp.full((16,), j * 16 + h, jnp.int32)])
                    for i in range(vregs_per_head):
                        off = (h * vregs_per_head + i) * 16
                        plsc.addupdate(acc.at[lrow, pl.ds(off, 16)],
                                       gath[j, pl.ds(off, 16)] * sc)
                plsc.addupdate(acc.at[lrow, pl.ds(cdim, 16)],
                               hold[pl.ds(j * 16, 16)])

        def _drain_serial(nb):
            @pl.loop(0, nb)
            def _fire(fb):
                _issue_rec(fb, rec0, rsem0)
                _consume_rec(rec0, rsem0, hold0, gath0, gsem0).wait()

                @pl.loop(0, 16)
                def _scale(j):
                    lrow = ldst_l[pl.ds(fb * 16 + j, 16)][0]
                    for h in range(heads):
                        sc = plsc.load_gather(hold0, [jnp.full((16,), j * 16 + h, jnp.int32)])
                        for i in range(vregs_per_head):
                            off = (h * vregs_per_head + i) * 16
                            plsc.addupdate(acc.at[lrow, pl.ds(off, 16)],
                                           gath0[j, pl.ds(off, 16)] * sc)
                    plsc.addupdate(acc.at[lrow, pl.ds(cdim, 16)],
                                   hold0[pl.ds(j * 16, 16)])

        def _drain_piped(nb):
            npair = nb // 2

            @pl.when(nb > 0)
            def _():
                _issue_rec(0, rec0, rsem0)
                _issue_rec(1, rec1, rsem1)

            @pl.loop(0, npair)
            def _pair(k):
                b0 = 2 * k
                _consume_rec(rec0, rsem0, hold0, gath0, gsem0)

                @pl.when(b0 + 2 < nb)
                def _():
                    _issue_rec(b0 + 2, rec0, rsem0)

                @pl.when(k > 0)
                def _():
                    _process(b0 - 1, hold1, gath1, gsem1)

                _consume_rec(rec1, rsem1, hold1, gath1, gsem1)

                @pl.when(b0 + 3 < nb)
                def _():
                    _issue_rec(b0 + 3, rec1, rsem1)

                _process(b0, hold0, gath0, gsem0)

            @pl.when(nb > 0)
            def _():
                _process(nb - 1, hold1, gath1, gsem1)

        @pl.loop(0, n_sweep)
        def _sweep(s):
            lo = (s * 32 + wid) * rows_tec

            @pl.loop(0, rows_tec)
            def _zr(r):
                @pl.loop(0, nvr)
                def _zc(cc, _r=r):
                    acc[_r, pl.ds(cc * 16, 16)] = jnp.zeros((16,),
                                                            jnp.float32)

            def _chunk(c, cnt):
                base = c * CHUNK
                pltpu.sync_copy(dst_hbm.at[pl.ds(base, CHUNK)], dst_v)

                def _grp(g, cnt):
                    d16 = dst_v[pl.ds(g * 16, 16)]
                    m = (d16 >= lo) & (d16 < lo + rows_tec)
                    eid = base + g * 16 + iota16
                    plsc.store_compressed(eid_l.at[pl.ds(cnt, 16)], eid,
                                          mask=m)
                    plsc.store_compressed(ldst_l.at[pl.ds(cnt, 16)],
                                          d16 - lo, mask=m)
                    pc = plsc.all_reduce_population_count(m)
                    return cnt + pc[0]

                cnt = pl.loop(0, n_groups, init_carry=cnt, unroll=4)(_grp)

                nb_mid = jnp.where(cnt >= THRESH, cnt // 16, 0)
                _drain_serial(nb_mid)
                ev = eid_l[pl.ds(nb_mid * 16, 16)]
                lv = ldst_l[pl.ds(nb_mid * 16, 16)]
                eid_l[pl.ds(0, 16)] = ev
                ldst_l[pl.ds(0, 16)] = lv
                return cnt - nb_mid * 16

            cnt = pl.loop(0, n_chunks, init_carry=jnp.int32(0))(_chunk)

            eid_l[pl.ds(cnt, 16)] = jnp.full((16,), N_EDGES, jnp.int32)
            eid_l[pl.ds(cnt + 16, 16)] = jnp.full((16,), N_EDGES, jnp.int32)
            ldst_l[pl.ds(cnt, 16)] = jnp.zeros((16,), jnp.int32)
            ldst_l[pl.ds(cnt + 16, 16)] = jnp.zeros((16,), jnp.int32)
            _drain_piped(((cnt + 31) // 32) * 2)

            pltpu.sync_copy(acc, out_hbm.at[pl.ds(lo, rows_tec)])

    return pl.kernel(
        body,
        out_type=jax.ShapeDtypeStruct((N_PAD, row_w), jnp.float32),
        mesh=_mesh,
        compiler_params=_sc_params,
        scratch_types=[
            pltpu.VMEM((CHUNK,), jnp.int32),
            pltpu.VMEM((LCAP,), jnp.int32),
            pltpu.VMEM((LCAP,), jnp.int32),
            pltpu.VMEM((16, 128), jnp.float32),
            pltpu.VMEM((16, 128), jnp.float32),
            pltpu.VMEM((256,), jnp.float32),
            pltpu.VMEM((256,), jnp.float32),
            pltpu.VMEM((16, cdim), jnp.float32),
            pltpu.VMEM((16, cdim), jnp.float32),
            pltpu.VMEM((rows_tec, row_w), jnp.float32),
            pltpu.SemaphoreType.DMA,
            pltpu.SemaphoreType.DMA,
            pltpu.SemaphoreType.DMA,
            pltpu.SemaphoreType.DMA,
        ],
    )


_phaseC1 = _phaseC_make(C1, HEADS, ROW1, 64, 5)
_phaseC2 = _phaseC_make(C2, 1, ROW2, 160, 2)


def kernel(x, edge_index, edge_attr, W1, aS1, aD1, We1, aE1, b1, g1, be1,
           W2, aS2, aD2, We2, aE2, b2, g2, be2):
    f32 = jnp.float32
    src = edge_index[0].astype(jnp.int32)
    dst = edge_index[1].astype(jnp.int32)

    x_pad = jnp.zeros((N_PAD, 64), f32).at[:N_NODES, :D_IN].set(x)
    W1p = jnp.zeros((64, C1), f32).at[:D_IN].set(W1)
    e_pad = E_PAD - N_EDGES
    srcB = jnp.concatenate([src, jnp.zeros((e_pad,), jnp.int32)])
    dstB = jnp.concatenate([dst, jnp.zeros((e_pad,), jnp.int32)])
    dstC = jnp.concatenate([dst, jnp.full((e_pad,), SENTINEL, jnp.int32)])
    eaT_pad = jnp.zeros((8, E_PAD), f32).at[:D_EDGE, :N_EDGES].set(
        edge_attr.T)
    WeA1 = (We1.reshape(D_EDGE, HEADS, HID) * aE1).sum(-1)
    WeA2 = (We2.reshape(D_EDGE, 1, HID) * aE2).sum(-1)
    WeApack = jnp.zeros((8, 8), f32)
    WeApack = WeApack.at[:HEADS, :D_EDGE].set(WeA1.T)
    WeApack = WeApack.at[HEADS, :D_EDGE].set(WeA2[:, 0])
    WaS2 = (W2.reshape(C1, 1, HID) * aS2).sum(-1)
    WaD2 = (W2.reshape(C1, 1, HID) * aD2).sum(-1)
    W2aug = jnp.zeros((C1, 384), f32)
    W2aug = W2aug.at[:, :C2].set(W2)
    W2aug = W2aug.at[:, C2:C2 + 1].set(WaS2)
    W2aug = W2aug.at[:, C2 + 1:C2 + 2].set(WaD2)

    aS1r = aS1[0]
    aD1r = aD1[0]
    b1r = b1.reshape(1, C1)
    g1r = g1.reshape(1, C1)
    be1r = be1.reshape(1, C1)
    b2r = b2.reshape(1, C2)
    g2r = g2.reshape(1, C2)
    be2r = be2.reshape(1, C2)

    h1, nod1 = _tc1(x_pad, W1p, aS1r, aD1r)
    aeT = _tce(WeApack, eaT_pad).reshape(-1)
    rec1 = _phaseB1(nod1.reshape(-1), srcB, dstB, aeT).reshape(E_PAD, 128)
    out_un1 = _phaseC1(h1, dstC, rec1)
    h1n, aux1 = _post1(out_un1, h1, aS1r, aD1r, b1r, g1r, be1r)

    h2t, nod2 = _tc2(h1n, W2aug, aux1)
    rec2 = _phaseB2(nod2.reshape(-1), srcB, dstB, aeT).reshape(E_PAD, 128)
    out_un2 = _phaseC2(h2t, dstC, rec2)
    out = _post2(out_un2, h2t, nod2, b2r, g2r, be2r)

    return out[:N_NODES]
